# baseline (speedup 1.0000x reference)
import functools

import jax
import jax.numpy as jnp
from jax import lax
from jax.experimental import pallas as pl
from jax.experimental.pallas import tpu as pltpu
from jax.experimental.pallas import tpu_sc as plsc

CH = 128
NBUF = 7
K = 5


@functools.cache
def _build_gather(B: int, D: int):
    info = plsc.get_sparse_core_info()
    NC, NS = info.num_cores, info.num_subcores
    NW = NC * NS
    assert B % NW == 0, (B, NW)
    bpw = B // NW
    assert bpw % CH == 0, (bpw, CH)
    nch = bpw // CH
    head = NBUF
    tail = next(t for t in range(K, K + NBUF) if (nch - head - t) % NBUF == 0)
    assert nch >= head + tail

    mesh = plsc.VectorSubcoreMesh(core_axis_name="c", subcore_axis_name="s")

    def body(idx_hbm, tab_hbm, out_hbm, idx_v, bufs, gsems, ssems):
        wid = lax.axis_index("s") * NC + lax.axis_index("c")
        base = wid * bpw
        pltpu.sync_copy(idx_hbm.at[pl.ds(base, bpw)], idx_v)

        def g_desc(j, b):
            return pltpu.make_async_copy(
                tab_hbm.at[idx_v.at[pl.ds(j * CH, CH)]], bufs.at[b], gsems.at[b]
            )

        def s_desc(j, b):
            return pltpu.make_async_copy(
                bufs.at[b], out_hbm.at[pl.ds(base + j * CH, CH)], ssems.at[b]
            )

        def step(j, t, store_wait, next_gather):
            g_desc(j, t).wait()
            s_desc(j, t).start()
            if next_gather:
                b2 = (t + K) % NBUF
                if store_wait:
                    s_desc(j + K - NBUF, b2).wait()
                g_desc(j + K, b2).start()

        for j in range(K):
            g_desc(j, j % NBUF).start()
        for j in range(head):
            step(j, j % NBUF, j >= NBUF - K, j + K < nch)

        @pl.loop(head, nch - tail, step=NBUF)
        def _(j0):
            for t in range(NBUF):
                step(j0 + t, t, True, True)

        for j in range(nch - tail, nch):
            step(j, j % NBUF, j >= NBUF - K, j + K < nch)
        for j in range(nch - NBUF, nch):
            s_desc(j, j % NBUF).wait()

    return pl.kernel(
        body,
        out_type=jax.ShapeDtypeStruct((B, D), jnp.float32),
        mesh=mesh,
        scratch_types=[
            pltpu.VMEM((bpw,), jnp.int32),
            pltpu.VMEM((NBUF, CH, D), jnp.float32),
            pltpu.SemaphoreType.DMA((NBUF,)),
            pltpu.SemaphoreType.DMA((NBUF,)),
        ],
    )


def kernel(x, table):
    NI, JW = x.shape
    B = NI * JW
    idx = x.T.astype(jnp.int32).reshape(B)
    out2d = _build_gather(B, table.shape[1])(idx, table)
    return out2d.reshape(JW, NI, table.shape[1]).transpose(1, 0, 2)
